# baseline (speedup 1.0000x reference)
import functools

import jax
import jax.numpy as jnp
from jax import lax
from jax.experimental import pallas as pl
from jax.experimental.pallas import tpu as pltpu
from jax.experimental.pallas import tpu_sc as plsc

D_EMB = 1000
ROWS = 4096
COLS = 20
N = ROWS * COLS
NC, NS, L = 2, 16, 16
NW = NC * NS
RPW = N // NW
ZWORDS = 512_000
NZDMA = RPW * D_EMB // ZWORDS
ZINIT = ZWORDS // NS
IPT = 128
NIDMA = RPW // IPT

_mesh = plsc.VectorSubcoreMesh(core_axis_name="c", subcore_axis_name="s")


@functools.partial(
    pl.kernel,
    mesh=_mesh,
    out_type=jax.ShapeDtypeStruct((N * D_EMB,), jnp.float32),
    scratch_types=[
        pltpu.VMEM((ZINIT,), jnp.float32),
        pltpu.VMEM_SHARED((ZWORDS,), jnp.float32),
        pltpu.VMEM((RPW,), jnp.int32),
        pltpu.VMEM((NIDMA, IPT), jnp.int32),
        pltpu.VMEM((IPT,), jnp.float32),
        pltpu.SemaphoreType.DMA,
        pltpu.SemaphoreType.DMA,
    ],
    compiler_params=pltpu.CompilerParams(needs_layout_passes=False),
)
def _sc_onehot(x_hbm, out_hbm, zb, szero, xall, offs, fives, sem1, sem2):
    sid = lax.axis_index("s")
    wid = sid * NC + lax.axis_index("c")
    gbase = wid * RPW

    zeros16 = jnp.zeros((L,), jnp.float32)
    lane = lax.iota(jnp.int32, L)

    def zbody(k, carry):
        for u in range(8):
            zb[pl.ds((k * 8 + u) * L, L)] = zeros16
        return carry

    lax.fori_loop(0, ZINIT // (8 * L), zbody, 0)
    pltpu.sync_copy(zb, szero.at[pl.ds(sid * ZINIT, ZINIT)])
    plsc.subcore_barrier()

    pltpu.sync_copy(x_hbm.at[pl.ds(gbase, RPW)], xall)
    for u in range(IPT // L):
        fives[pl.ds(u * L, L)] = jnp.full((L,), 5.0, jnp.float32)

    def obody(k, carry):
        j = k // (IPT // L)
        c = (k % (IPT // L)) * L
        xv = xall[pl.ds(k * L, L)]
        offs[j, pl.ds(c, L)] = (gbase + k * L + lane) * D_EMB + xv
        return carry

    lax.fori_loop(0, RPW // L, obody, 0)

    zhandles = []
    for k in range(NZDMA):
        zhandles.append(
            pltpu.async_copy(
                szero,
                out_hbm.at[pl.ds(gbase * D_EMB + k * ZWORDS, ZWORDS)],
                sem1,
            )
        )
    for h in zhandles:
        h.wait()

    ihandles = []
    for j in range(NIDMA):
        ihandles.append(
            pltpu.async_copy(fives, out_hbm.at[offs.at[j]], sem2)
        )
    for h in ihandles:
        h.wait()


def kernel(x):
    flat = _sc_onehot(x.reshape(N))
    return flat.reshape(ROWS, COLS, D_EMB)


# speedup vs baseline: 7.5997x; 7.5997x over previous
import jax
import jax.numpy as jnp
from jax.experimental import pallas as pl
from jax.experimental.pallas import tpu as pltpu

D_EMB = 1000
ROWS = 4096
COLS = 20
CBLK = 128


def _onehot_block(x_ref, o_ref):
    xb = jnp.transpose(x_ref[...])
    iota = jax.lax.broadcasted_iota(jnp.int32, (COLS, D_EMB, CBLK), 1)
    o_ref[...] = jnp.where(xb[:, None, :] == iota, 5.0, 0.0).astype(jnp.float32)


def kernel(x):
    out_t = pl.pallas_call(
        _onehot_block,
        grid=(ROWS // CBLK,),
        in_specs=[pl.BlockSpec((CBLK, COLS), lambda i: (i, 0))],
        out_specs=pl.BlockSpec((COLS, D_EMB, CBLK), lambda i: (0, 0, i)),
        out_shape=jax.ShapeDtypeStruct((COLS, D_EMB, ROWS), jnp.float32),
        compiler_params=pltpu.CompilerParams(
            dimension_semantics=("parallel",)),
    )(x)
    return out_t.transpose(2, 0, 1)


# speedup vs baseline: 7.6650x; 1.0086x over previous
import jax
import jax.numpy as jnp
from jax.experimental import pallas as pl
from jax.experimental.pallas import tpu as pltpu

D_EMB = 1000
ROWS = 4096
COLS = 20
VBLK = 40


def _onehot_block(xt_ref, o_ref):
    i = pl.program_id(0)
    xb = xt_ref[...]
    iota = jax.lax.broadcasted_iota(jnp.int32, (COLS, VBLK, ROWS), 1) + i * VBLK
    o_ref[...] = jnp.where(xb[:, None, :] == iota, 5.0, 0.0).astype(jnp.float32)


def kernel(x):
    xt = x.T
    out_t = pl.pallas_call(
        _onehot_block,
        grid=(D_EMB // VBLK,),
        in_specs=[pl.BlockSpec((COLS, ROWS), lambda i: (0, 0))],
        out_specs=pl.BlockSpec((COLS, VBLK, ROWS), lambda i: (0, i, 0)),
        out_shape=jax.ShapeDtypeStruct((COLS, D_EMB, ROWS), jnp.float32),
        compiler_params=pltpu.CompilerParams(
            dimension_semantics=("parallel",)),
    )(xt)
    return out_t.transpose(2, 0, 1)


# speedup vs baseline: 7.8171x; 1.0198x over previous
import jax
import jax.numpy as jnp
from jax.experimental import pallas as pl
from jax.experimental.pallas import tpu as pltpu

D_EMB = 1000
ROWS = 4096
COLS = 20
CBLK = 128


def _onehot_block(xt_ref, o_ref):
    xb = xt_ref[...]
    iota = jax.lax.broadcasted_iota(jnp.int32, (COLS, D_EMB, CBLK), 1)
    o_ref[...] = jnp.where(xb[:, None, :] == iota, 5.0, 0.0).astype(jnp.float32)


def kernel(x):
    xt = x.T
    out_t = pl.pallas_call(
        _onehot_block,
        grid=(ROWS // CBLK,),
        in_specs=[pl.BlockSpec((COLS, CBLK), lambda i: (0, i))],
        out_specs=pl.BlockSpec((COLS, D_EMB, CBLK), lambda i: (0, 0, i)),
        out_shape=jax.ShapeDtypeStruct((COLS, D_EMB, ROWS), jnp.float32),
        compiler_params=pltpu.CompilerParams(
            dimension_semantics=("parallel",)),
    )(xt)
    return out_t.transpose(2, 0, 1)
